# baseline (speedup 1.0000x reference)
import jax
import jax.numpy as jnp
from jax import lax
from jax.experimental import pallas as pl
from jax.experimental.pallas import tpu as pltpu
from jax.experimental.pallas import tpu_sc as plsc

_N = 10000
_E = 160000
_C = 256
_L = 16
_NC = 2
_NS = 16
_B = 128
_EPAD = 163840
_EPS = _EPAD // _NS
_NBLK = _EPS // _B
_G = _EPAD // _B
_UPC = 8
_RPS = 624
_TAIL = _N - _NS * _RPS
_ZR = 104
_FB = 64
_FEPS = _EPAD // (_NC * _NS)
_FNB = _FEPS // _FB

_mesh = plsc.VectorSubcoreMesh(core_axis_name="c", subcore_axis_name="s")


def _spmm_body(t_hbm, meta_hbm, lv_hbm, zz_hbm, out_hbm,
               acc, meta_v, lv_v, x_v, msem, gsem, ssem):
    cid = lax.axis_index("c")
    sid = lax.axis_index("s")
    e0 = sid * _EPS
    g0 = e0 // _B
    r0 = sid * _RPS
    last = sid == _NS - 1

    def issue_meta(q, g, m):
        gg = g0 + g
        pltpu.async_copy(meta_hbm.at[gg], meta_v.at[m], msem)
        pltpu.async_copy(lv_hbm.at[q, :, pl.ds(gg * _B, _B)], lv_v.at[m],
                         msem)

    def wait_meta(q, g, m):
        gg = g0 + g
        pltpu.make_async_copy(meta_hbm.at[gg], meta_v.at[m], msem).wait()
        pltpu.make_async_copy(lv_hbm.at[q, :, pl.ds(gg * _B, _B)],
                              lv_v.at[m], msem).wait()

    def issue_gather(g_, p, m):
        h = _B // 2
        pltpu.async_copy(t_hbm.at[g_].at[meta_v.at[m, 0, pl.ds(0, h)]],
                         x_v.at[p, pl.ds(0, h)], gsem)
        pltpu.async_copy(t_hbm.at[g_].at[meta_v.at[m, 0, pl.ds(h, h)]],
                         x_v.at[p, pl.ds(h, h)], gsem)

    def wait_gather(g_, p, m):
        h = _B // 2
        pltpu.make_async_copy(t_hbm.at[g_].at[meta_v.at[m, 0, pl.ds(0, h)]],
                              x_v.at[p, pl.ds(0, h)], gsem).wait()
        pltpu.make_async_copy(t_hbm.at[g_].at[meta_v.at[m, 0, pl.ds(h, h)]],
                              x_v.at[p, pl.ds(h, h)], gsem).wait()

    def compute(p, m):
        def grp(t, _):
            base = t * _L
            lvv = lv_v[m, 0, pl.ds(base, _L)]
            for k in range(_L):
                e = base + k
                s = lvv[k]
                for j in range(8):
                    sl = pl.ds(j * _L, _L)
                    x_v[p, e, sl] = s * x_v[p, e, sl]
            return 0
        lax.fori_loop(0, _B // _L, grp, 0)

    def issue_scatter(p, m):
        pltpu.async_copy(x_v.at[p], acc.at[meta_v.at[m, 1]], ssem, add=True)

    def wait_scatter():
        pltpu.make_async_copy(x_v.at[0], acc.at[meta_v.at[0, 1]],
                              ssem).wait()

    def unit_body(u, _):
        gu = cid * _UPC + u
        g_ = gu // 4
        q = gu % 4
        pltpu.sync_copy(zz_hbm.at[pl.ds(0, _RPS)], acc.at[pl.ds(r0, _RPS)])

        @pl.when(last)
        def _():
            pltpu.sync_copy(zz_hbm.at[pl.ds(0, _TAIL)],
                            acc.at[pl.ds(_N - _TAIL, _TAIL)])
        plsc.subcore_barrier()

        issue_meta(q, 0, 0)
        wait_meta(q, 0, 0)
        issue_gather(g_, 0, 0)
        issue_meta(q, 1, 1)

        def quad(j, _):
            b0 = 4 * j
            for k in range(4):
                bk = b0 + k
                p = k % 2
                m = k
                mn = (k + 1) % 4
                wait_gather(g_, p, m)
                compute(p, m)
                issue_scatter(p, m)
                if k == 0:
                    @pl.when(j > 0)
                    def _():
                        wait_scatter()
                else:
                    wait_scatter()
                if k < 3:
                    wait_meta(q, bk + 1, mn)
                    issue_gather(g_, 1 - p, mn)
                else:
                    @pl.when(j < _NBLK // 4 - 1)
                    def _():
                        wait_meta(q, bk + 1, mn)
                        issue_gather(g_, 1 - p, mn)
                if k < 2:
                    issue_meta(q, bk + 2, (k + 2) % 4)
                else:
                    @pl.when(j < _NBLK // 4 - 1)
                    def _():
                        issue_meta(q, bk + 2, (k + 2) % 4)
            return 0
        lax.fori_loop(0, _NBLK // 4, quad, 0)
        wait_scatter()

        plsc.subcore_barrier()
        pltpu.sync_copy(acc.at[pl.ds(r0, _RPS)],
                        out_hbm.at[gu, pl.ds(r0, _RPS), :])

        @pl.when(last)
        def _():
            t0 = _N - _TAIL
            pltpu.sync_copy(acc.at[pl.ds(t0, _TAIL)],
                            out_hbm.at[gu, pl.ds(t0, _TAIL), :])
        plsc.subcore_barrier()
        return 0

    lax.fori_loop(0, _UPC, unit_body, 0)


def _spmm(tbl, meta2, lv3, zz):
    f = pl.kernel(
        _spmm_body,
        out_type=jax.ShapeDtypeStruct((16, _N, 128), jnp.float32),
        mesh=_mesh,
        scratch_types=[
            pltpu.VMEM_SHARED((_N, 128), jnp.float32),
            pltpu.VMEM((4, 2, _B), jnp.int32),
            pltpu.VMEM((4, 1, _B), jnp.float32),
            pltpu.VMEM((2, _B, 128), jnp.float32),
            pltpu.SemaphoreType.DMA,
            pltpu.SemaphoreType.DMA,
            pltpu.SemaphoreType.DMA,
        ],
    )
    return f(tbl, meta2, lv3, zz)


def _combine_body(acc_ref, w_ref, b_ref, r_ref, i_ref):
    a = acc_ref[...]
    w0 = w_ref[0]
    w1 = w_ref[1]

    def d(x, w):
        return jnp.dot(x, w, preferred_element_type=jnp.float32)

    real = (d(a[0], w0) - d(a[1], w0)) + (d(a[4], w1) - d(a[5], w1))
    imag = (d(a[2], w0) + d(a[3], w0)) + (d(a[6], w1) + d(a[7], w1))
    real = real + b_ref[...]
    imag = imag + b_ref[...]
    m = (real >= 0).astype(jnp.float32)
    r_ref[...] = m * real
    i_ref[...] = m * imag


def _combine(acc, w, b):
    blk = 1000
    return pl.pallas_call(
        _combine_body,
        grid=(_N // blk,),
        in_specs=[
            pl.BlockSpec((8, blk, _C), lambda i: (0, i, 0)),
            pl.BlockSpec((2, _C, _C), lambda i: (0, 0, 0)),
            pl.BlockSpec((1, _C), lambda i: (0, 0)),
        ],
        out_specs=[pl.BlockSpec((blk, _C), lambda i: (i, 0))] * 2,
        out_shape=[jax.ShapeDtypeStruct((_N, _C), jnp.float32)] * 2,
    )(acc, w, b)


def _uv_body(r_ref, i_ref, lw_ref, lb_ref, u_ref, v_ref):
    r = r_ref[...]
    im = i_ref[...]

    def d(x, w):
        return jnp.dot(x, w, preferred_element_type=jnp.float32)

    u_ref[...] = d(r, lw_ref[0:_C]) + d(im, lw_ref[2 * _C:3 * _C]) + lb_ref[...]
    v_ref[...] = d(r, lw_ref[_C:2 * _C]) + d(im, lw_ref[3 * _C:4 * _C])


def _uv(r, im, lin_w, lin_b):
    blk = 1000
    return pl.pallas_call(
        _uv_body,
        grid=(_N // blk,),
        in_specs=[
            pl.BlockSpec((blk, _C), lambda i: (i, 0)),
            pl.BlockSpec((blk, _C), lambda i: (i, 0)),
            pl.BlockSpec((4 * _C, _C), lambda i: (0, 0)),
            pl.BlockSpec((1, _C), lambda i: (0, 0)),
        ],
        out_specs=[pl.BlockSpec((blk, _C), lambda i: (i, 0))] * 2,
        out_shape=[jax.ShapeDtypeStruct((_N, _C), jnp.float32)] * 2,
    )(r, im, lin_w, lin_b)


def _final_body(uv_hbm, fm_hbm, out_hbm,
                fi_v, ul_v, uh_v, vl_v, vh_v, o_v, msem, gsem):
    cid = lax.axis_index("c")
    sid = lax.axis_index("s")
    wid = cid * _NS + sid
    e0 = wid * _FEPS
    g0 = e0 // _FB

    def issue_meta(g, p):
        pltpu.async_copy(fm_hbm.at[g0 + g], fi_v.at[p], msem)

    def wait_meta(g, p):
        pltpu.make_async_copy(fm_hbm.at[g0 + g], fi_v.at[p], msem).wait()

    def issue_gather(p):
        pltpu.async_copy(uv_hbm.at[0].at[fi_v.at[p, 0]], ul_v.at[p], gsem)
        pltpu.async_copy(uv_hbm.at[1].at[fi_v.at[p, 0]], uh_v.at[p], gsem)
        pltpu.async_copy(uv_hbm.at[2].at[fi_v.at[p, 1]], vl_v.at[p], gsem)
        pltpu.async_copy(uv_hbm.at[3].at[fi_v.at[p, 1]], vh_v.at[p], gsem)

    def wait_gather(p):
        pltpu.make_async_copy(uv_hbm.at[0].at[fi_v.at[p, 0]], ul_v.at[p],
                              gsem).wait()
        pltpu.make_async_copy(uv_hbm.at[1].at[fi_v.at[p, 0]], uh_v.at[p],
                              gsem).wait()
        pltpu.make_async_copy(uv_hbm.at[2].at[fi_v.at[p, 1]], vl_v.at[p],
                              gsem).wait()
        pltpu.make_async_copy(uv_hbm.at[3].at[fi_v.at[p, 1]], vh_v.at[p],
                              gsem).wait()

    def compute_store(g, p):
        def grp(e, _):
            for j in range(8):
                sl = pl.ds(j * _L, _L)
                o_v[e, 0, sl] = ul_v[p, e, sl] + vl_v[p, e, sl]
                o_v[e, 1, sl] = uh_v[p, e, sl] + vh_v[p, e, sl]
            return 0
        lax.fori_loop(0, _FB, grp, 0)
        eb = e0 + g * _FB
        pltpu.sync_copy(o_v, out_hbm.at[pl.ds(eb, _FB)])

    issue_meta(0, 0)
    wait_meta(0, 0)
    issue_gather(0)

    def pair(i, _):
        ga = 2 * i
        gb = 2 * i + 1
        issue_meta(gb, 1)
        wait_gather(0)
        wait_meta(gb, 1)
        issue_gather(1)
        compute_store(ga, 0)

        @pl.when(i < _FNB // 2 - 1)
        def _():
            issue_meta(ga + 2, 0)
        wait_gather(1)

        @pl.when(i < _FNB // 2 - 1)
        def _():
            wait_meta(ga + 2, 0)
            issue_gather(0)
        compute_store(gb, 1)
        return 0
    lax.fori_loop(0, _FNB // 2, pair, 0)


def _final(uv, fmeta):
    f = pl.kernel(
        _final_body,
        out_type=jax.ShapeDtypeStruct((_EPAD, 2, 128), jnp.float32),
        mesh=_mesh,
        scratch_types=[
            pltpu.VMEM((2, 2, _FB), jnp.int32),
            pltpu.VMEM((2, _FB, 128), jnp.float32),
            pltpu.VMEM((2, _FB, 128), jnp.float32),
            pltpu.VMEM((2, _FB, 128), jnp.float32),
            pltpu.VMEM((2, _FB, 128), jnp.float32),
            pltpu.VMEM((_FB, 2, 128), jnp.float32),
            pltpu.SemaphoreType.DMA,
            pltpu.SemaphoreType.DMA,
        ],
    )
    return f(uv, fmeta)


def kernel(real, imag, L_real_values, L_imag_values, weight1, bias1, weight2,
           bias2, lin_w, lin_b, edge_index, index):
    ei = edge_index.astype(jnp.int32)
    rows = ei[0]
    cols = ei[1]
    pad = _EPAD - _E
    rows_p = jnp.concatenate([rows, jnp.zeros((pad,), jnp.int32)])
    cols_p = jnp.concatenate([cols, jnp.zeros((pad,), jnp.int32)])
    zpad = jnp.zeros((4, pad), jnp.float32)
    lv = jnp.concatenate([
        jnp.stack([L_real_values[0], L_imag_values[0],
                   L_real_values[1], L_imag_values[1]]), zpad,
    ], axis=1)
    lv3 = lv.reshape(4, 1, _EPAD)
    meta2 = jnp.stack([cols_p.reshape(_G, _B),
                       rows_p.reshape(_G, _B)], axis=1)
    zz = jnp.zeros((_RPS, 128), jnp.float32)

    idx = index.astype(jnp.int32)
    i0p = jnp.concatenate([idx[:, 0], jnp.zeros((pad,), jnp.int32)])
    i1p = jnp.concatenate([idx[:, 1], jnp.zeros((pad,), jnp.int32)])
    fmeta = jnp.stack([i0p.reshape(_EPAD // _FB, _FB),
                       i1p.reshape(_EPAD // _FB, _FB)], axis=1)

    b1 = bias1.reshape(1, _C)
    b2 = bias2.reshape(1, _C)
    lbr = lin_b.reshape(1, _C)

    def layer(xr, xi, w, b):
        tbl = jnp.concatenate(
            [xr.reshape(_N, 4, 64), xi.reshape(_N, 4, 64)], axis=2
        ).transpose(1, 0, 2)
        out16 = _spmm(tbl, meta2, lv3, zz)
        o = out16.reshape(4, 4, _N, 2, 64)
        def asm(q, h):
            return o[:, q, :, h, :].transpose(1, 0, 2).reshape(_N, _C)
        acc = jnp.stack([
            asm(0, 0), asm(1, 1), asm(1, 0), asm(0, 1),
            asm(2, 0), asm(3, 1), asm(3, 0), asm(2, 1),
        ])
        return _combine(acc, w, b)

    r, im = layer(real, imag, weight1, b1)
    r, im = layer(r, im, weight2, b2)
    u, v = _uv(r, im, lin_w, lbr)
    uv = jnp.stack([u[:, :128], u[:, 128:], v[:, :128], v[:, 128:]])
    out2 = _final(uv, fmeta)
    return out2.reshape(_EPAD, _C)[:_E]


# speedup vs baseline: 1.1592x; 1.1592x over previous
import jax
import jax.numpy as jnp
from jax import lax
from jax.experimental import pallas as pl
from jax.experimental.pallas import tpu as pltpu
from jax.experimental.pallas import tpu_sc as plsc

_N = 10000
_E = 160000
_C = 256
_L = 16
_NC = 2
_NS = 16
_B = 128
_EPAD = 163840
_EPS = _EPAD // _NS
_NBLK = _EPS // _B
_G = _EPAD // _B
_UPC = 8
_RPS = 624
_TAIL = _N - _NS * _RPS
_ZR = 104
_FB = 64
_FEPS = _EPAD // (_NC * _NS)
_FNB = _FEPS // _FB

_mesh = plsc.VectorSubcoreMesh(core_axis_name="c", subcore_axis_name="s")


def _spmm_body(t_hbm, meta_hbm, lv_hbm, zz_hbm, out_hbm,
               acc, meta_v, lv_v, x_v, msem, gsem, ssem):
    cid = lax.axis_index("c")
    sid = lax.axis_index("s")
    e0 = sid * _EPS
    g0 = e0 // _B
    r0 = sid * _RPS
    last = sid == _NS - 1

    def issue_meta(q, g, m):
        gg = g0 + g
        pltpu.async_copy(meta_hbm.at[gg], meta_v.at[m], msem)
        pltpu.async_copy(lv_hbm.at[q, :, pl.ds(gg * _B, _B)], lv_v.at[m],
                         msem)

    def wait_meta(q, g, m):
        gg = g0 + g
        pltpu.make_async_copy(meta_hbm.at[gg], meta_v.at[m], msem).wait()
        pltpu.make_async_copy(lv_hbm.at[q, :, pl.ds(gg * _B, _B)],
                              lv_v.at[m], msem).wait()

    def issue_gather(g_, p, m):
        h = _B // 2
        pltpu.async_copy(t_hbm.at[g_].at[meta_v.at[m, 0, pl.ds(0, h)]],
                         x_v.at[p, pl.ds(0, h)], gsem)
        pltpu.async_copy(t_hbm.at[g_].at[meta_v.at[m, 0, pl.ds(h, h)]],
                         x_v.at[p, pl.ds(h, h)], gsem)

    def wait_gather(g_, p, m):
        h = _B // 2
        pltpu.make_async_copy(t_hbm.at[g_].at[meta_v.at[m, 0, pl.ds(0, h)]],
                              x_v.at[p, pl.ds(0, h)], gsem).wait()
        pltpu.make_async_copy(t_hbm.at[g_].at[meta_v.at[m, 0, pl.ds(h, h)]],
                              x_v.at[p, pl.ds(h, h)], gsem).wait()

    def compute(p, m):
        def grp(t, _):
            base = t * _L
            lvv = lv_v[m, 0, pl.ds(base, _L)]
            for k in range(_L):
                e = base + k
                s = lvv[k]
                for j in range(8):
                    sl = pl.ds(j * _L, _L)
                    x_v[p, e, sl] = s * x_v[p, e, sl]
            return 0
        lax.fori_loop(0, _B // _L, grp, 0)

    def issue_scatter(p, m):
        pltpu.async_copy(x_v.at[p], acc.at[meta_v.at[m, 1]], ssem, add=True)

    def wait_scatter():
        pltpu.make_async_copy(x_v.at[0], acc.at[meta_v.at[0, 1]],
                              ssem).wait()

    def unit_body(u, _):
        gu = cid * _UPC + u
        g_ = gu // 4
        q = gu % 4
        pltpu.sync_copy(zz_hbm.at[pl.ds(0, _RPS)], acc.at[pl.ds(r0, _RPS)])

        @pl.when(last)
        def _():
            pltpu.sync_copy(zz_hbm.at[pl.ds(0, _TAIL)],
                            acc.at[pl.ds(_N - _TAIL, _TAIL)])
        plsc.subcore_barrier()

        issue_meta(q, 0, 0)
        wait_meta(q, 0, 0)
        issue_gather(g_, 0, 0)
        issue_meta(q, 1, 1)

        def quad(j, _):
            b0 = 4 * j
            for k in range(4):
                bk = b0 + k
                p = k % 2
                m = k
                mn = (k + 1) % 4
                wait_gather(g_, p, m)
                compute(p, m)
                issue_scatter(p, m)
                if k == 0:
                    @pl.when(j > 0)
                    def _():
                        wait_scatter()
                else:
                    wait_scatter()
                if k < 3:
                    wait_meta(q, bk + 1, mn)
                    issue_gather(g_, 1 - p, mn)
                else:
                    @pl.when(j < _NBLK // 4 - 1)
                    def _():
                        wait_meta(q, bk + 1, mn)
                        issue_gather(g_, 1 - p, mn)
                if k < 2:
                    issue_meta(q, bk + 2, (k + 2) % 4)
                else:
                    @pl.when(j < _NBLK // 4 - 1)
                    def _():
                        issue_meta(q, bk + 2, (k + 2) % 4)
            return 0
        lax.fori_loop(0, _NBLK // 4, quad, 0)
        wait_scatter()

        plsc.subcore_barrier()
        pltpu.sync_copy(acc.at[pl.ds(r0, _RPS)],
                        out_hbm.at[gu, pl.ds(r0, _RPS), :])

        @pl.when(last)
        def _():
            t0 = _N - _TAIL
            pltpu.sync_copy(acc.at[pl.ds(t0, _TAIL)],
                            out_hbm.at[gu, pl.ds(t0, _TAIL), :])
        plsc.subcore_barrier()
        return 0

    lax.fori_loop(0, _UPC, unit_body, 0)


def _spmm(tbl, meta2, lv3, zz):
    f = pl.kernel(
        _spmm_body,
        out_type=jax.ShapeDtypeStruct((16, _N, 128), jnp.float32),
        mesh=_mesh,
        scratch_types=[
            pltpu.VMEM_SHARED((_N, 128), jnp.float32),
            pltpu.VMEM((4, 2, _B), jnp.int32),
            pltpu.VMEM((4, 1, _B), jnp.float32),
            pltpu.VMEM((2, _B, 128), jnp.float32),
            pltpu.SemaphoreType.DMA,
            pltpu.SemaphoreType.DMA,
            pltpu.SemaphoreType.DMA,
        ],
    )
    return f(tbl, meta2, lv3, zz)


def _combine_body(o_ref, w_ref, b_ref, r_ref, i_ref):
    o = o_ref[...]
    w0 = w_ref[0]
    w1 = w_ref[1]

    def asm(q, h):
        return jnp.concatenate(
            [o[4 * g + q, :, 64 * h:64 * (h + 1)] for g in range(4)], axis=1)

    def d(x, w):
        return jnp.dot(x, w, preferred_element_type=jnp.float32)

    real = (d(asm(0, 0), w0) - d(asm(1, 1), w0)) + (
        d(asm(2, 0), w1) - d(asm(3, 1), w1))
    imag = (d(asm(1, 0), w0) + d(asm(0, 1), w0)) + (
        d(asm(3, 0), w1) + d(asm(2, 1), w1))
    real = real + b_ref[...]
    imag = imag + b_ref[...]
    m = (real >= 0).astype(jnp.float32)
    r_ref[...] = m * real
    i_ref[...] = m * imag


def _combine(out16, w, b):
    blk = 1000
    return pl.pallas_call(
        _combine_body,
        grid=(_N // blk,),
        in_specs=[
            pl.BlockSpec((16, blk, 128), lambda i: (0, i, 0)),
            pl.BlockSpec((2, _C, _C), lambda i: (0, 0, 0)),
            pl.BlockSpec((1, _C), lambda i: (0, 0)),
        ],
        out_specs=[pl.BlockSpec((blk, _C), lambda i: (i, 0))] * 2,
        out_shape=[jax.ShapeDtypeStruct((_N, _C), jnp.float32)] * 2,
    )(out16, w, b)


def _uv_body(r_ref, i_ref, lw_ref, lb_ref, u_ref, v_ref):
    r = r_ref[...]
    im = i_ref[...]

    def d(x, w):
        return jnp.dot(x, w, preferred_element_type=jnp.float32)

    u_ref[...] = d(r, lw_ref[0:_C]) + d(im, lw_ref[2 * _C:3 * _C]) + lb_ref[...]
    v_ref[...] = d(r, lw_ref[_C:2 * _C]) + d(im, lw_ref[3 * _C:4 * _C])


def _uv(r, im, lin_w, lin_b):
    blk = 1000
    return pl.pallas_call(
        _uv_body,
        grid=(_N // blk,),
        in_specs=[
            pl.BlockSpec((blk, _C), lambda i: (i, 0)),
            pl.BlockSpec((blk, _C), lambda i: (i, 0)),
            pl.BlockSpec((4 * _C, _C), lambda i: (0, 0)),
            pl.BlockSpec((1, _C), lambda i: (0, 0)),
        ],
        out_specs=[pl.BlockSpec((blk, _C), lambda i: (i, 0))] * 2,
        out_shape=[jax.ShapeDtypeStruct((_N, _C), jnp.float32)] * 2,
    )(r, im, lin_w, lin_b)


def _final_body(uv_hbm, fm_hbm, out_hbm,
                fi_v, ul_v, uh_v, vl_v, vh_v, o_v, msem, gsem):
    cid = lax.axis_index("c")
    sid = lax.axis_index("s")
    wid = cid * _NS + sid
    e0 = wid * _FEPS
    g0 = e0 // _FB

    def issue_meta(g, p):
        pltpu.async_copy(fm_hbm.at[g0 + g], fi_v.at[p], msem)

    def wait_meta(g, p):
        pltpu.make_async_copy(fm_hbm.at[g0 + g], fi_v.at[p], msem).wait()

    def issue_gather(p):
        pltpu.async_copy(uv_hbm.at[0].at[fi_v.at[p, 0]], ul_v.at[p], gsem)
        pltpu.async_copy(uv_hbm.at[1].at[fi_v.at[p, 0]], uh_v.at[p], gsem)
        pltpu.async_copy(uv_hbm.at[2].at[fi_v.at[p, 1]], vl_v.at[p], gsem)
        pltpu.async_copy(uv_hbm.at[3].at[fi_v.at[p, 1]], vh_v.at[p], gsem)

    def wait_gather(p):
        pltpu.make_async_copy(uv_hbm.at[0].at[fi_v.at[p, 0]], ul_v.at[p],
                              gsem).wait()
        pltpu.make_async_copy(uv_hbm.at[1].at[fi_v.at[p, 0]], uh_v.at[p],
                              gsem).wait()
        pltpu.make_async_copy(uv_hbm.at[2].at[fi_v.at[p, 1]], vl_v.at[p],
                              gsem).wait()
        pltpu.make_async_copy(uv_hbm.at[3].at[fi_v.at[p, 1]], vh_v.at[p],
                              gsem).wait()

    def compute_store(g, p):
        def grp(e, _):
            for j in range(8):
                sl = pl.ds(j * _L, _L)
                o_v[e, 0, sl] = ul_v[p, e, sl] + vl_v[p, e, sl]
                o_v[e, 1, sl] = uh_v[p, e, sl] + vh_v[p, e, sl]
            return 0
        lax.fori_loop(0, _FB, grp, 0)
        eb = e0 + g * _FB
        pltpu.sync_copy(o_v, out_hbm.at[pl.ds(eb, _FB)])

    issue_meta(0, 0)
    wait_meta(0, 0)
    issue_gather(0)

    def pair(i, _):
        ga = 2 * i
        gb = 2 * i + 1
        issue_meta(gb, 1)
        wait_gather(0)
        wait_meta(gb, 1)
        issue_gather(1)
        compute_store(ga, 0)

        @pl.when(i < _FNB // 2 - 1)
        def _():
            issue_meta(ga + 2, 0)
        wait_gather(1)

        @pl.when(i < _FNB // 2 - 1)
        def _():
            wait_meta(ga + 2, 0)
            issue_gather(0)
        compute_store(gb, 1)
        return 0
    lax.fori_loop(0, _FNB // 2, pair, 0)


def _final(uv, fmeta):
    f = pl.kernel(
        _final_body,
        out_type=jax.ShapeDtypeStruct((_EPAD, 2, 128), jnp.float32),
        mesh=_mesh,
        scratch_types=[
            pltpu.VMEM((2, 2, _FB), jnp.int32),
            pltpu.VMEM((2, _FB, 128), jnp.float32),
            pltpu.VMEM((2, _FB, 128), jnp.float32),
            pltpu.VMEM((2, _FB, 128), jnp.float32),
            pltpu.VMEM((2, _FB, 128), jnp.float32),
            pltpu.VMEM((_FB, 2, 128), jnp.float32),
            pltpu.SemaphoreType.DMA,
            pltpu.SemaphoreType.DMA,
        ],
    )
    return f(uv, fmeta)


def kernel(real, imag, L_real_values, L_imag_values, weight1, bias1, weight2,
           bias2, lin_w, lin_b, edge_index, index):
    ei = edge_index.astype(jnp.int32)
    rows = ei[0]
    cols = ei[1]
    pad = _EPAD - _E
    rows_p = jnp.concatenate([rows, jnp.zeros((pad,), jnp.int32)])
    cols_p = jnp.concatenate([cols, jnp.zeros((pad,), jnp.int32)])
    zpad = jnp.zeros((4, pad), jnp.float32)
    lv = jnp.concatenate([
        jnp.stack([L_real_values[0], L_imag_values[0],
                   L_real_values[1], L_imag_values[1]]), zpad,
    ], axis=1)
    lv3 = lv.reshape(4, 1, _EPAD)
    meta2 = jnp.stack([cols_p.reshape(_G, _B),
                       rows_p.reshape(_G, _B)], axis=1)
    zz = jnp.zeros((_RPS, 128), jnp.float32)

    idx = index.astype(jnp.int32)
    i0p = jnp.concatenate([idx[:, 0], jnp.zeros((pad,), jnp.int32)])
    i1p = jnp.concatenate([idx[:, 1], jnp.zeros((pad,), jnp.int32)])
    fmeta = jnp.stack([i0p.reshape(_EPAD // _FB, _FB),
                       i1p.reshape(_EPAD // _FB, _FB)], axis=1)

    b1 = bias1.reshape(1, _C)
    b2 = bias2.reshape(1, _C)
    lbr = lin_b.reshape(1, _C)

    def layer(xr, xi, w, b):
        tbl = jnp.concatenate(
            [xr.reshape(_N, 4, 64), xi.reshape(_N, 4, 64)], axis=2
        ).transpose(1, 0, 2)
        out16 = _spmm(tbl, meta2, lv3, zz)
        return _combine(out16, w, b)

    r, im = layer(real, imag, weight1, b1)
    r, im = layer(r, im, weight2, b2)
    u, v = _uv(r, im, lin_w, lbr)
    uv = jnp.stack([u[:, :128], u[:, 128:], v[:, :128], v[:, 128:]])
    out2 = _final(uv, fmeta)
    return out2.reshape(_EPAD, _C)[:_E]
